# baseline (speedup 1.0000x reference)
import functools

import jax
import jax.numpy as jnp
from jax import lax
from jax.experimental import pallas as pl
from jax.experimental.pallas import tpu as pltpu
from jax.experimental.pallas import tpu_sc as plsc

N = 10000
NP = 10240
D_IN = 128
HID = 64
HHID = 32
E = 320000
NTILES = 32
NSUB = 16
CH = 128
CH2 = 256
CHUNKS32 = 79
CHUNKS16 = 79
EP = NTILES * CHUNKS32 * CH
RPT = NP // NSUB
NBUF = 4

_mesh = plsc.VectorSubcoreMesh(core_axis_name="c", subcore_axis_name="s")
_sc_params = pltpu.CompilerParams(use_tc_tiling_on_sc=False)


def _zero_vmem_2d(ref, rows, cols):
    z16 = jnp.zeros((16,), jnp.float32)

    def body(i, _):
        for k in range(cols // 16):
            ref[i, pl.ds(k * 16, 16)] = z16
        return 0

    lax.fori_loop(0, rows, body, 0)


def _zero_vmem_1d(ref, n):
    z16 = jnp.zeros((16,), jnp.float32)

    def body(i, _):
        ref[pl.ds(i * 16, 16)] = z16
        return 0

    lax.fori_loop(0, n // 16, body, 0)


def _gather_scatter_ring(g_src, src_v, dst_v, acc_sh, bufs, gsems, ssems,
                         nchunks):
    gd = [None] * NBUF
    sd = [None] * NBUF
    for j in range(min(NBUF, nchunks)):
        gd[j] = pltpu.async_copy(g_src.at[src_v.at[j]], bufs[j], gsems[j])
    for j in range(nchunks):
        b = j % NBUF
        gd[b].wait()
        sd[b] = pltpu.async_copy(bufs[b], acc_sh.at[dst_v.at[j]], ssems[b],
                                 add=True)
        jn = j + NBUF
        if jn < nchunks:
            sd[b].wait()
            sd[b] = None
            gd[b] = pltpu.async_copy(g_src.at[src_v.at[jn]], bufs[b], gsems[b])
    for b in range(NBUF):
        if sd[b] is not None:
            sd[b].wait()


def _deg_body(dsts_hbm, out_hbm, dst_v, ones_v, zbuf, acc_sh, *ssems):
    c = lax.axis_index("c")
    s = lax.axis_index("s")
    wid = c * NSUB + s
    idx_d = pltpu.async_copy(dsts_hbm.at[wid], dst_v, ssems[0])

    one16 = jnp.ones((16,), jnp.float32)
    for k in range(CH // 16):
        ones_v[pl.ds(k * 16, 16)] = one16

    row0 = s * RPT
    _zero_vmem_1d(zbuf, RPT)
    pltpu.sync_copy(zbuf, acc_sh.at[pl.ds(row0, RPT)])
    idx_d.wait()
    plsc.subcore_barrier()

    sd = [None] * NBUF
    for j in range(CHUNKS32):
        b = j % NBUF
        if sd[b] is not None:
            sd[b].wait()
        sd[b] = pltpu.async_copy(ones_v, acc_sh.at[dst_v.at[j]], ssems[b],
                                 add=True)
    for b in range(NBUF):
        if sd[b] is not None:
            sd[b].wait()
    plsc.subcore_barrier()
    pltpu.sync_copy(acc_sh.at[pl.ds(row0, RPT)],
                    out_hbm.at[c].at[pl.ds(row0, RPT)])


_deg_kernel = functools.partial(
    pl.kernel,
    out_type=jax.ShapeDtypeStruct((2, NP), jnp.float32),
    mesh=_mesh,
    compiler_params=_sc_params,
    scratch_types=[
        pltpu.VMEM((CHUNKS32, CH), jnp.int32),
        pltpu.VMEM((CH,), jnp.float32),
        pltpu.VMEM((RPT,), jnp.float32),
        pltpu.VMEM_SHARED((NP,), jnp.float32),
    ] + [pltpu.SemaphoreType.DMA] * NBUF,
)(_deg_body)


def _prop64_body(g_hbm, srcs_hbm, dsts_hbm, out_hbm,
                 src_v, dst_v, b0, b1, b2, b3, zbuf, g_sh, acc_sh, *sems):
    c = lax.axis_index("c")
    s = lax.axis_index("s")
    idx_s = pltpu.async_copy(srcs_hbm.at[s], src_v, sems[NBUF])
    idx_d = pltpu.async_copy(dsts_hbm.at[s], dst_v, sems[NBUF + 1])

    row0 = s * RPT
    stg = []
    for k in range(RPT // CH):
        stg.append(pltpu.async_copy(
            g_hbm.at[c].at[pl.ds(row0 + k * CH, CH)],
            g_sh.at[pl.ds(row0 + k * CH, CH)], sems[k % NBUF]))
    _zero_vmem_2d(zbuf, CH, HHID)
    for k in range(RPT // CH):
        pltpu.sync_copy(zbuf, acc_sh.at[pl.ds(row0 + k * CH, CH)])
    for d in stg:
        d.wait()
    idx_s.wait()
    idx_d.wait()
    plsc.subcore_barrier()

    _gather_scatter_ring(g_sh, src_v, dst_v, acc_sh,
                         (b0, b1, b2, b3), sems[:NBUF], sems[NBUF:],
                         CHUNKS16)
    plsc.subcore_barrier()
    pltpu.sync_copy(acc_sh.at[pl.ds(row0, RPT)],
                    out_hbm.at[c].at[pl.ds(row0, RPT)])


_prop64_kernel = functools.partial(
    pl.kernel,
    out_type=jax.ShapeDtypeStruct((2, NP, HHID), jnp.float32),
    mesh=_mesh,
    compiler_params=_sc_params,
    scratch_types=[
        pltpu.VMEM((CHUNKS16, CH2), jnp.int32),
        pltpu.VMEM((CHUNKS16, CH2), jnp.int32),
        pltpu.VMEM((CH2, HHID), jnp.float32),
        pltpu.VMEM((CH2, HHID), jnp.float32),
        pltpu.VMEM((CH2, HHID), jnp.float32),
        pltpu.VMEM((CH2, HHID), jnp.float32),
        pltpu.VMEM((CH, HHID), jnp.float32),
        pltpu.VMEM_SHARED((NP, HHID), jnp.float32),
        pltpu.VMEM_SHARED((NP, HHID), jnp.float32),
    ] + [pltpu.SemaphoreType.DMA] * (2 * NBUF),
)(_prop64_body)


def _prop1_body(g_hbm, srcs_hbm, dsts_hbm, out_hbm,
                src_v, dst_v, b0, b1, b2, b3, zbuf, g_sh, acc_sh, *sems):
    c = lax.axis_index("c")
    s = lax.axis_index("s")
    wid = c * NSUB + s
    idx_s = pltpu.async_copy(srcs_hbm.at[wid], src_v, sems[NBUF])
    idx_d = pltpu.async_copy(dsts_hbm.at[wid], dst_v, sems[NBUF + 1])

    row0 = s * RPT
    stg = pltpu.async_copy(g_hbm.at[pl.ds(row0, RPT)],
                           g_sh.at[pl.ds(row0, RPT)], sems[0])
    _zero_vmem_1d(zbuf, RPT)
    pltpu.sync_copy(zbuf, acc_sh.at[pl.ds(row0, RPT)])
    stg.wait()
    idx_s.wait()
    idx_d.wait()
    plsc.subcore_barrier()

    _gather_scatter_ring(g_sh, src_v, dst_v, acc_sh,
                         (b0, b1, b2, b3), sems[:NBUF], sems[NBUF:],
                         CHUNKS32)
    plsc.subcore_barrier()
    pltpu.sync_copy(acc_sh.at[pl.ds(row0, RPT)],
                    out_hbm.at[c].at[pl.ds(row0, RPT)])


_prop1_kernel = functools.partial(
    pl.kernel,
    out_type=jax.ShapeDtypeStruct((2, NP), jnp.float32),
    mesh=_mesh,
    compiler_params=_sc_params,
    scratch_types=[
        pltpu.VMEM((CHUNKS32, CH), jnp.int32),
        pltpu.VMEM((CHUNKS32, CH), jnp.int32),
        pltpu.VMEM((CH,), jnp.float32),
        pltpu.VMEM((CH,), jnp.float32),
        pltpu.VMEM((CH,), jnp.float32),
        pltpu.VMEM((CH,), jnp.float32),
        pltpu.VMEM((RPT,), jnp.float32),
        pltpu.VMEM_SHARED((NP,), jnp.float32),
        pltpu.VMEM_SHARED((NP,), jnp.float32),
    ] + [pltpu.SemaphoreType.DMA] * (2 * NBUF),
)(_prop1_body)


BR = 1024


def _tc_mm1_body(x_ref, w_ref, h_ref):
    h_ref[...] = jnp.dot(x_ref[...], w_ref[...],
                         preferred_element_type=jnp.float32)


def _tc_mm1(xp, W1):
    return pl.pallas_call(
        _tc_mm1_body,
        grid=(NP // BR,),
        in_specs=[
            pl.BlockSpec((BR, D_IN), lambda i: (i, 0)),
            pl.BlockSpec((D_IN, HID), lambda i: (0, 0)),
        ],
        out_specs=pl.BlockSpec((BR, HID), lambda i: (i, 0)),
        out_shape=jax.ShapeDtypeStruct((NP, HID), jnp.float32),
    )(xp, W1)


def _tca_body(h_ref, d0_ref, d1_ref, g_ref, dis_ref):
    deg = 1.0 + d0_ref[...] + d1_ref[...]
    dis = lax.rsqrt(deg)
    g = dis * h_ref[...]
    g_ref[0] = g[:, :HHID]
    g_ref[1] = g[:, HHID:]
    dis_ref[...] = dis


def _tc_stage_a(h1, d0, d1):
    return pl.pallas_call(
        _tca_body,
        grid=(NP // BR,),
        in_specs=[
            pl.BlockSpec((BR, HID), lambda i: (i, 0)),
            pl.BlockSpec((BR, 1), lambda i: (i, 0)),
            pl.BlockSpec((BR, 1), lambda i: (i, 0)),
        ],
        out_specs=[
            pl.BlockSpec((2, BR, HHID), lambda i: (0, i, 0)),
            pl.BlockSpec((BR, 1), lambda i: (i, 0)),
        ],
        out_shape=[
            jax.ShapeDtypeStruct((2, NP, HHID), jnp.float32),
            jax.ShapeDtypeStruct((NP, 1), jnp.float32),
        ],
    )(h1, d0, d1)


def _tcb_body(p_ref, g_ref, dis_ref, w_ref, b_ref, gout_ref):
    dis = dis_ref[...]
    al = jnp.maximum(dis * (p_ref[0] + g_ref[0]) + b_ref[:, :HHID], 0.0)
    ar = jnp.maximum(dis * (p_ref[1] + g_ref[1]) + b_ref[:, HHID:], 0.0)
    a = jnp.concatenate([al, ar], axis=1)
    g = dis * jnp.dot(a, w_ref[...], preferred_element_type=jnp.float32)
    gout_ref[0] = g[:, :HHID]
    gout_ref[1] = g[:, HHID:]


def _tc_stage_mid(p, g, dis, W, b):
    return pl.pallas_call(
        _tcb_body,
        grid=(NP // BR,),
        in_specs=[
            pl.BlockSpec((2, BR, HHID), lambda i: (0, i, 0)),
            pl.BlockSpec((2, BR, HHID), lambda i: (0, i, 0)),
            pl.BlockSpec((BR, 1), lambda i: (i, 0)),
            pl.BlockSpec((HID, HID), lambda i: (0, 0)),
            pl.BlockSpec((1, HID), lambda i: (0, 0)),
        ],
        out_specs=pl.BlockSpec((2, BR, HHID), lambda i: (0, i, 0)),
        out_shape=jax.ShapeDtypeStruct((2, NP, HHID), jnp.float32),
    )(p, g, dis, W, b)


def _tcc_body(p_ref, g_ref, dis_ref, w_ref, b_ref, gout_ref):
    dis = dis_ref[...]
    al = jnp.maximum(dis * (p_ref[0] + g_ref[0]) + b_ref[:, :HHID], 0.0)
    ar = jnp.maximum(dis * (p_ref[1] + g_ref[1]) + b_ref[:, HHID:], 0.0)
    a = jnp.concatenate([al, ar], axis=1)
    gout_ref[...] = dis * jnp.dot(a, w_ref[...],
                                  preferred_element_type=jnp.float32)


def _tc_stage_last(p, g, dis, W3, b):
    return pl.pallas_call(
        _tcc_body,
        grid=(NP // BR,),
        in_specs=[
            pl.BlockSpec((2, BR, HHID), lambda i: (0, i, 0)),
            pl.BlockSpec((2, BR, HHID), lambda i: (0, i, 0)),
            pl.BlockSpec((BR, 1), lambda i: (i, 0)),
            pl.BlockSpec((HID, 1), lambda i: (0, 0)),
            pl.BlockSpec((1, HID), lambda i: (0, 0)),
        ],
        out_specs=pl.BlockSpec((BR, 1), lambda i: (i, 0)),
        out_shape=jax.ShapeDtypeStruct((NP, 1), jnp.float32),
    )(p, g, dis, W3, b)


def _tcd_body(q0_ref, q1_ref, g3_ref, dis_ref, b3_ref, out_ref):
    out_ref[...] = dis_ref[...] * (q0_ref[...] + q1_ref[...] + g3_ref[...]) + b3_ref[0, 0]


def _tc_stage_out(q0, q1, g3, dis, b3):
    return pl.pallas_call(
        _tcd_body,
        out_shape=jax.ShapeDtypeStruct((NP // 128, 128), jnp.float32),
    )(q0, q1, g3, dis, b3)


def kernel(x, edge_index, W1, b1, W2, b2, W3, b3):
    eip = jnp.pad(edge_index.astype(jnp.int32), ((0, 0), (0, EP - E)),
                  constant_values=NP - 1)
    src16 = eip[0].reshape(NSUB, CHUNKS16, CH2)
    dst16 = eip[1].reshape(NSUB, CHUNKS16, CH2)
    src32 = eip[0].reshape(NTILES, CHUNKS32, CH)
    dst32 = eip[1].reshape(NTILES, CHUNKS32, CH)
    xp = jnp.pad(x, ((0, NP - N), (0, 0)))

    h1 = _tc_mm1(xp, W1)
    degp = _deg_kernel(dst32)
    d0 = degp[0].reshape(NP, 1)
    d1 = degp[1].reshape(NP, 1)

    g1, dis = _tc_stage_a(h1, d0, d1)
    p1 = _prop64_kernel(g1, src16, dst16)
    g2 = _tc_stage_mid(p1, g1, dis, W2, b1.reshape(1, HID))
    p2 = _prop64_kernel(g2, src16, dst16)
    g3 = _tc_stage_last(p2, g2, dis, W3, b2.reshape(1, HID))
    p3 = _prop1_kernel(g3.reshape(NP), src32, dst32)
    out = _tc_stage_out(p3[0].reshape(NP // 128, 128),
                        p3[1].reshape(NP // 128, 128),
                        g3.reshape(NP // 128, 128),
                        dis.reshape(NP // 128, 128), b3.reshape(1, 1))
    return out.reshape(NP, 1)[:N]


# speedup vs baseline: 1.0228x; 1.0228x over previous
import functools

import jax
import jax.numpy as jnp
from jax import lax
from jax.experimental import pallas as pl
from jax.experimental.pallas import tpu as pltpu
from jax.experimental.pallas import tpu_sc as plsc

N = 10000
NP = 10240
D_IN = 128
HID = 64
HHID = 32
E = 320000
NTILES = 32
NSUB = 16
CH = 128
CHUNKS32 = 79
CHUNKS16 = 158
EP = NTILES * CHUNKS32 * CH
RPT = NP // NSUB
NBUF = 4

_mesh = plsc.VectorSubcoreMesh(core_axis_name="c", subcore_axis_name="s")
_sc_params = pltpu.CompilerParams(use_tc_tiling_on_sc=False)


def _zero_vmem_2d(ref, rows, cols):
    z16 = jnp.zeros((16,), jnp.float32)

    def body(i, _):
        for k in range(cols // 16):
            ref[i, pl.ds(k * 16, 16)] = z16
        return 0

    lax.fori_loop(0, rows, body, 0)


def _zero_vmem_1d(ref, n):
    z16 = jnp.zeros((16,), jnp.float32)

    def body(i, _):
        ref[pl.ds(i * 16, 16)] = z16
        return 0

    lax.fori_loop(0, n // 16, body, 0)


def _gather_scatter_ring(g_src, src_v, dst_v, acc_sh, bufs, gsems, ssems,
                         nchunks):
    gd = [None] * NBUF
    sd = [None] * NBUF
    for j in range(min(NBUF, nchunks)):
        gd[j] = pltpu.async_copy(g_src.at[src_v.at[j]], bufs[j], gsems[j])
    for j in range(nchunks):
        b = j % NBUF
        gd[b].wait()
        sd[b] = pltpu.async_copy(bufs[b], acc_sh.at[dst_v.at[j]], ssems[b],
                                 add=True)
        jn = j + NBUF
        if jn < nchunks:
            sd[b].wait()
            sd[b] = None
            gd[b] = pltpu.async_copy(g_src.at[src_v.at[jn]], bufs[b], gsems[b])
    for b in range(NBUF):
        if sd[b] is not None:
            sd[b].wait()


def _deg_body(dsts_hbm, out_hbm, dst_v, ones_v, zbuf, acc_sh, *ssems):
    c = lax.axis_index("c")
    s = lax.axis_index("s")
    wid = c * NSUB + s
    idx_d = pltpu.async_copy(dsts_hbm.at[wid], dst_v, ssems[0])

    one16 = jnp.ones((16,), jnp.float32)
    for k in range(CH // 16):
        ones_v[pl.ds(k * 16, 16)] = one16

    row0 = s * RPT
    _zero_vmem_1d(zbuf, RPT)
    pltpu.sync_copy(zbuf, acc_sh.at[pl.ds(row0, RPT)])
    idx_d.wait()
    plsc.subcore_barrier()

    sd = [None] * NBUF
    for j in range(CHUNKS32):
        b = j % NBUF
        if sd[b] is not None:
            sd[b].wait()
        sd[b] = pltpu.async_copy(ones_v, acc_sh.at[dst_v.at[j]], ssems[b],
                                 add=True)
    for b in range(NBUF):
        if sd[b] is not None:
            sd[b].wait()
    plsc.subcore_barrier()
    pltpu.sync_copy(acc_sh.at[pl.ds(row0, RPT)],
                    out_hbm.at[c].at[pl.ds(row0, RPT)])


_deg_kernel = functools.partial(
    pl.kernel,
    out_type=jax.ShapeDtypeStruct((2, NP), jnp.float32),
    mesh=_mesh,
    compiler_params=_sc_params,
    scratch_types=[
        pltpu.VMEM((CHUNKS32, CH), jnp.int32),
        pltpu.VMEM((CH,), jnp.float32),
        pltpu.VMEM((RPT,), jnp.float32),
        pltpu.VMEM_SHARED((NP,), jnp.float32),
    ] + [pltpu.SemaphoreType.DMA] * NBUF,
)(_deg_body)


def _prop64_body(g_hbm, srcs_hbm, dsts_hbm, out_hbm,
                 src_v, dst_v, b0, b1, b2, b3, zbuf, g_sh, acc_sh, *sems):
    c = lax.axis_index("c")
    s = lax.axis_index("s")
    idx_s = pltpu.async_copy(srcs_hbm.at[s], src_v, sems[NBUF])
    idx_d = pltpu.async_copy(dsts_hbm.at[s], dst_v, sems[NBUF + 1])

    row0 = s * RPT
    stg = []
    for k in range(RPT // CH):
        stg.append(pltpu.async_copy(
            g_hbm.at[c].at[pl.ds(row0 + k * CH, CH)],
            g_sh.at[pl.ds(row0 + k * CH, CH)], sems[k % NBUF]))
    _zero_vmem_2d(zbuf, CH, HHID)
    for k in range(RPT // CH):
        pltpu.sync_copy(zbuf, acc_sh.at[pl.ds(row0 + k * CH, CH)])
    for d in stg:
        d.wait()
    idx_s.wait()
    idx_d.wait()
    plsc.subcore_barrier()

    _gather_scatter_ring(g_sh, src_v, dst_v, acc_sh,
                         (b0, b1, b2, b3), sems[:NBUF], sems[NBUF:],
                         CHUNKS16)
    plsc.subcore_barrier()
    pltpu.sync_copy(acc_sh.at[pl.ds(row0, RPT)],
                    out_hbm.at[c].at[pl.ds(row0, RPT)])


_prop64_kernel = functools.partial(
    pl.kernel,
    out_type=jax.ShapeDtypeStruct((2, NP, HHID), jnp.float32),
    mesh=_mesh,
    compiler_params=_sc_params,
    scratch_types=[
        pltpu.VMEM((CHUNKS16, CH), jnp.int32),
        pltpu.VMEM((CHUNKS16, CH), jnp.int32),
        pltpu.VMEM((CH, HHID), jnp.float32),
        pltpu.VMEM((CH, HHID), jnp.float32),
        pltpu.VMEM((CH, HHID), jnp.float32),
        pltpu.VMEM((CH, HHID), jnp.float32),
        pltpu.VMEM((CH, HHID), jnp.float32),
        pltpu.VMEM_SHARED((NP, HHID), jnp.float32),
        pltpu.VMEM_SHARED((NP, HHID), jnp.float32),
    ] + [pltpu.SemaphoreType.DMA] * (2 * NBUF),
)(_prop64_body)


def _prop1_body(g_hbm, srcs_hbm, dsts_hbm, out_hbm,
                src_v, dst_v, b0, b1, b2, b3, zbuf, g_sh, acc_sh, *sems):
    c = lax.axis_index("c")
    s = lax.axis_index("s")
    wid = c * NSUB + s
    idx_s = pltpu.async_copy(srcs_hbm.at[wid], src_v, sems[NBUF])
    idx_d = pltpu.async_copy(dsts_hbm.at[wid], dst_v, sems[NBUF + 1])

    row0 = s * RPT
    stg = pltpu.async_copy(g_hbm.at[pl.ds(row0, RPT)],
                           g_sh.at[pl.ds(row0, RPT)], sems[0])
    _zero_vmem_1d(zbuf, RPT)
    pltpu.sync_copy(zbuf, acc_sh.at[pl.ds(row0, RPT)])
    stg.wait()
    idx_s.wait()
    idx_d.wait()
    plsc.subcore_barrier()

    _gather_scatter_ring(g_sh, src_v, dst_v, acc_sh,
                         (b0, b1, b2, b3), sems[:NBUF], sems[NBUF:],
                         CHUNKS32)
    plsc.subcore_barrier()
    pltpu.sync_copy(acc_sh.at[pl.ds(row0, RPT)],
                    out_hbm.at[c].at[pl.ds(row0, RPT)])


_prop1_kernel = functools.partial(
    pl.kernel,
    out_type=jax.ShapeDtypeStruct((2, NP), jnp.float32),
    mesh=_mesh,
    compiler_params=_sc_params,
    scratch_types=[
        pltpu.VMEM((CHUNKS32, CH), jnp.int32),
        pltpu.VMEM((CHUNKS32, CH), jnp.int32),
        pltpu.VMEM((CH,), jnp.float32),
        pltpu.VMEM((CH,), jnp.float32),
        pltpu.VMEM((CH,), jnp.float32),
        pltpu.VMEM((CH,), jnp.float32),
        pltpu.VMEM((RPT,), jnp.float32),
        pltpu.VMEM_SHARED((NP,), jnp.float32),
        pltpu.VMEM_SHARED((NP,), jnp.float32),
    ] + [pltpu.SemaphoreType.DMA] * (2 * NBUF),
)(_prop1_body)


BR = 1024


def _tca_body(x_ref, w_ref, d0_ref, d1_ref, g_ref, dis_ref):
    h = jnp.dot(x_ref[...], w_ref[...], preferred_element_type=jnp.float32)
    deg = 1.0 + d0_ref[...] + d1_ref[...]
    dis = lax.rsqrt(deg)
    g = dis * h
    g_ref[0] = g[:, :HHID]
    g_ref[1] = g[:, HHID:]
    dis_ref[...] = dis


def _tc_stage_a(xp, W1, d0, d1):
    return pl.pallas_call(
        _tca_body,
        grid=(NP // BR,),
        in_specs=[
            pl.BlockSpec((BR, D_IN), lambda i: (i, 0)),
            pl.BlockSpec((D_IN, HID), lambda i: (0, 0)),
            pl.BlockSpec((BR, 1), lambda i: (i, 0)),
            pl.BlockSpec((BR, 1), lambda i: (i, 0)),
        ],
        out_specs=[
            pl.BlockSpec((2, BR, HHID), lambda i: (0, i, 0)),
            pl.BlockSpec((BR, 1), lambda i: (i, 0)),
        ],
        out_shape=[
            jax.ShapeDtypeStruct((2, NP, HHID), jnp.float32),
            jax.ShapeDtypeStruct((NP, 1), jnp.float32),
        ],
    )(xp, W1, d0, d1)


def _tcb_body(p_ref, g_ref, dis_ref, w_ref, b_ref, gout_ref):
    dis = dis_ref[...]
    al = jnp.maximum(dis * (p_ref[0] + g_ref[0]) + b_ref[:, :HHID], 0.0)
    ar = jnp.maximum(dis * (p_ref[1] + g_ref[1]) + b_ref[:, HHID:], 0.0)
    a = jnp.concatenate([al, ar], axis=1)
    g = dis * jnp.dot(a, w_ref[...], preferred_element_type=jnp.float32)
    gout_ref[0] = g[:, :HHID]
    gout_ref[1] = g[:, HHID:]


def _tc_stage_mid(p, g, dis, W, b):
    return pl.pallas_call(
        _tcb_body,
        grid=(NP // BR,),
        in_specs=[
            pl.BlockSpec((2, BR, HHID), lambda i: (0, i, 0)),
            pl.BlockSpec((2, BR, HHID), lambda i: (0, i, 0)),
            pl.BlockSpec((BR, 1), lambda i: (i, 0)),
            pl.BlockSpec((HID, HID), lambda i: (0, 0)),
            pl.BlockSpec((1, HID), lambda i: (0, 0)),
        ],
        out_specs=pl.BlockSpec((2, BR, HHID), lambda i: (0, i, 0)),
        out_shape=jax.ShapeDtypeStruct((2, NP, HHID), jnp.float32),
    )(p, g, dis, W, b)


def _tcc_body(p_ref, g_ref, dis_ref, w_ref, b_ref, gout_ref):
    dis = dis_ref[...]
    al = jnp.maximum(dis * (p_ref[0] + g_ref[0]) + b_ref[:, :HHID], 0.0)
    ar = jnp.maximum(dis * (p_ref[1] + g_ref[1]) + b_ref[:, HHID:], 0.0)
    a = jnp.concatenate([al, ar], axis=1)
    gout_ref[...] = dis * jnp.dot(a, w_ref[...],
                                  preferred_element_type=jnp.float32)


def _tc_stage_last(p, g, dis, W3, b):
    return pl.pallas_call(
        _tcc_body,
        grid=(NP // BR,),
        in_specs=[
            pl.BlockSpec((2, BR, HHID), lambda i: (0, i, 0)),
            pl.BlockSpec((2, BR, HHID), lambda i: (0, i, 0)),
            pl.BlockSpec((BR, 1), lambda i: (i, 0)),
            pl.BlockSpec((HID, 1), lambda i: (0, 0)),
            pl.BlockSpec((1, HID), lambda i: (0, 0)),
        ],
        out_specs=pl.BlockSpec((BR, 1), lambda i: (i, 0)),
        out_shape=jax.ShapeDtypeStruct((NP, 1), jnp.float32),
    )(p, g, dis, W3, b)


def _tcd_body(q0_ref, q1_ref, g3_ref, dis_ref, b3_ref, out_ref):
    out_ref[...] = dis_ref[...] * (q0_ref[...] + q1_ref[...] + g3_ref[...]) + b3_ref[0, 0]


def _tc_stage_out(q0, q1, g3, dis, b3):
    return pl.pallas_call(
        _tcd_body,
        out_shape=jax.ShapeDtypeStruct((NP // 128, 128), jnp.float32),
    )(q0, q1, g3, dis, b3)


def kernel(x, edge_index, W1, b1, W2, b2, W3, b3):
    eip = jnp.pad(edge_index.astype(jnp.int32), ((0, 0), (0, EP - E)),
                  constant_values=NP - 1)
    src16 = eip[0].reshape(NSUB, CHUNKS16, CH)
    dst16 = eip[1].reshape(NSUB, CHUNKS16, CH)
    src32 = eip[0].reshape(NTILES, CHUNKS32, CH)
    dst32 = eip[1].reshape(NTILES, CHUNKS32, CH)
    xp = jnp.pad(x, ((0, NP - N), (0, 0)))

    degp = _deg_kernel(dst32)
    d0 = degp[0].reshape(NP, 1)
    d1 = degp[1].reshape(NP, 1)

    g1, dis = _tc_stage_a(xp, W1, d0, d1)
    p1 = _prop64_kernel(g1, src16, dst16)
    g2 = _tc_stage_mid(p1, g1, dis, W2, b1.reshape(1, HID))
    p2 = _prop64_kernel(g2, src16, dst16)
    g3 = _tc_stage_last(p2, g2, dis, W3, b2.reshape(1, HID))
    p3 = _prop1_kernel(g3.reshape(NP), src32, dst32)
    out = _tc_stage_out(p3[0].reshape(NP // 128, 128),
                        p3[1].reshape(NP // 128, 128),
                        g3.reshape(NP // 128, 128),
                        dis.reshape(NP // 128, 128), b3.reshape(1, 1))
    return out.reshape(NP, 1)[:N]
